# baseline (speedup 1.0000x reference)
import functools

import jax
import jax.numpy as jnp
from jax import lax
from jax.experimental import pallas as pl
from jax.experimental.pallas import tpu as pltpu
from jax.experimental.pallas import tpu_sc as plsc

_NC = 2
_NS = 16
_NW = _NC * _NS
_CHUNK = 128
_PACK = 8


def _sc_gather(user_tp, item_tp, user_pidx, item_pidx):
    B = user_pidx.shape[0]
    W = user_tp.shape[1]
    bpw = B // _NW
    cpw = bpw // _CHUNK

    mesh = plsc.VectorSubcoreMesh(core_axis_name="c", subcore_axis_name="s")

    @functools.partial(
        pl.kernel,
        mesh=mesh,
        out_type=(jax.ShapeDtypeStruct((B, W), jnp.float32),
                  jax.ShapeDtypeStruct((B, W), jnp.float32)),
        scratch_types=[
            pltpu.VMEM((cpw, _CHUNK), jnp.int32),
            pltpu.VMEM((cpw, _CHUNK), jnp.int32),
            pltpu.VMEM((2, _CHUNK, 128), jnp.float32),
            pltpu.VMEM((2, _CHUNK, 128), jnp.float32),
            pltpu.SemaphoreType.DMA,
            pltpu.SemaphoreType.DMA,
            pltpu.SemaphoreType.DMA,
        ],
    )
    def sc_k(ut_hbm, it_hbm, ui_hbm, ii_hbm, uo_hbm, io_hbm,
             uidx_v, iidx_v, urows_v, irows_v, gsem_u, gsem_i, osem):
        wid = lax.axis_index("s") * _NC + lax.axis_index("c")
        pltpu.sync_copy(ui_hbm.at[pl.ds(wid * cpw, cpw)], uidx_v)
        pltpu.sync_copy(ii_hbm.at[pl.ds(wid * cpw, cpw)], iidx_v)

        def fire(c):
            b = c % 2
            return (pltpu.async_copy(ut_hbm.at[uidx_v.at[c]],
                                     urows_v.at[b], gsem_u),
                    pltpu.async_copy(it_hbm.at[iidx_v.at[c]],
                                     irows_v.at[b], gsem_i))

        g = [None] * cpw
        o = [None] * cpw
        g[0] = fire(0)
        for c in range(cpw):
            for h in g[c]:
                h.wait()
            b = c % 2
            row0 = wid * bpw + c * _CHUNK
            o[c] = (pltpu.async_copy(urows_v.at[b],
                                     uo_hbm.at[pl.ds(row0, _CHUNK)], osem),
                    pltpu.async_copy(irows_v.at[b],
                                     io_hbm.at[pl.ds(row0, _CHUNK)], osem))
            if c + 1 < cpw:
                if c >= 1:
                    for h in o[c - 1]:
                        h.wait()
                g[c + 1] = fire(c + 1)
        for c in (cpw - 2, cpw - 1):
            if c >= 0 and o[c] is not None:
                for h in o[c]:
                    h.wait()

    ui2 = user_pidx.reshape(B // _CHUNK, _CHUNK)
    ii2 = item_pidx.reshape(B // _CHUNK, _CHUNK)
    return sc_k(user_tp, item_tp, ui2, ii2)


def _select_sub(x, sel, D):
    acc = jnp.where(sel == 0, x[:, 0:D], 0.0)
    for p in range(1, _PACK):
        acc = acc + jnp.where(sel == p, x[:, p * D:(p + 1) * D], 0.0)
    return acc


def _mlp_body(xu_ref, xi_ref, su_ref, si_ref, w1u_ref, w1i_ref, b1_ref,
              w2_ref, b2_ref, w3_ref, b3_ref, wf_ref, bf_ref, o_ref):
    D = w1u_ref.shape[0]
    u = _select_sub(xu_ref[...], su_ref[...], D)
    it = _select_sub(xi_ref[...], si_ref[...], D)
    hp = jax.lax.Precision.HIGHEST
    h = jnp.dot(u, w1u_ref[...], precision=hp)
    h += jnp.dot(it, w1i_ref[...], precision=hp)
    h = jnp.maximum(h + b1_ref[...], 0.0)
    h = jnp.maximum(jnp.dot(h, w2_ref[...], precision=hp) + b2_ref[...], 0.0)
    h = jnp.maximum(jnp.dot(h, w3_ref[...], precision=hp) + b3_ref[...], 0.0)
    logits = jnp.dot(h, wf_ref[...], precision=hp) + bf_ref[...]
    o_ref[...] = jax.nn.sigmoid(logits)


def _tc_mlp(xu, xi, su, si, W1, b1, W2, b2, W3, b3, Wf, bf):
    B = xu.shape[0]
    M = W1.shape[0] // 2
    blk = 2048
    w1u, w1i = W1[:M], W1[M:]
    b1r, b2r, b3r = b1.reshape(1, -1), b2.reshape(1, -1), b3.reshape(1, -1)
    bfr = bf.reshape(1, 1)

    full = lambda shape: pl.BlockSpec(shape, lambda b: (0, 0))
    return pl.pallas_call(
        _mlp_body,
        grid=(B // blk,),
        in_specs=[
            pl.BlockSpec((blk, 128), lambda b: (b, 0)),
            pl.BlockSpec((blk, 128), lambda b: (b, 0)),
            pl.BlockSpec((blk, 1), lambda b: (b, 0)),
            pl.BlockSpec((blk, 1), lambda b: (b, 0)),
            full(w1u.shape), full(w1i.shape), full(b1r.shape),
            full(W2.shape), full(b2r.shape),
            full(W3.shape), full(b3r.shape),
            full(Wf.shape), full(bfr.shape),
        ],
        out_specs=pl.BlockSpec((blk, 1), lambda b: (b, 0)),
        out_shape=jax.ShapeDtypeStruct((B, 1), jnp.float32),
        compiler_params=pltpu.CompilerParams(
            dimension_semantics=("parallel",)),
    )(xu, xi, su, si, w1u, w1i, b1r, W2, b2r, W3, b3r, Wf, bfr)


def _relayout_body(xt_ref, o_ref):
    x = xt_ref[...]
    D, L = x.shape
    P = L // _PACK
    y = x.reshape(D, P, _PACK).transpose(1, 2, 0).reshape(P, _PACK * D)
    o_ref[...] = y


def _relayout(table_t):
    D, V = table_t.shape
    LBLK = 4096
    n = (V + LBLK - 1) // LBLK
    return pl.pallas_call(
        _relayout_body,
        grid=(n,),
        in_specs=[pl.BlockSpec((D, LBLK), lambda b: (0, b))],
        out_specs=pl.BlockSpec((LBLK // _PACK, _PACK * D), lambda b: (b, 0)),
        out_shape=jax.ShapeDtypeStruct((V * D // (_PACK * D), _PACK * D),
                                       jnp.float32),
        compiler_params=pltpu.CompilerParams(
            dimension_semantics=("parallel",)),
    )(table_t)


def kernel(user_input, item_input, user_table, item_table,
           W1, b1, W2, b2, W3, b3, Wf, bf):
    V, D = user_table.shape
    utp = _relayout(user_table.T)
    itp = _relayout(item_table.T)
    u_pidx = lax.shift_right_logical(user_input, 3)
    i_pidx = lax.shift_right_logical(item_input, 3)
    xu, xi = _sc_gather(utp, itp, u_pidx, i_pidx)
    su = (user_input & 7).astype(jnp.float32).reshape(-1, 1)
    si = (item_input & 7).astype(jnp.float32).reshape(-1, 1)
    return _tc_mlp(xu, xi, su, si, W1, b1, W2, b2, W3, b3, Wf, bf)


# speedup vs baseline: 12.2167x; 12.2167x over previous
import dataclasses
import functools

import jax
import jax.numpy as jnp
from jax import lax
from jax.experimental import pallas as pl
from jax.experimental.pallas import tpu as pltpu
from jax.experimental.pallas import tpu_sc as plsc

_NC = 2
_NS = 16
_NW = _NC * _NS
_NBUF = 8
_LANES = 128


def _sc_gather_t(user_t, item_t, user_idx, item_idx):
    D, V = user_t.shape
    B = user_idx.shape[0]
    bpw = B // _NW

    mesh = plsc.VectorSubcoreMesh(core_axis_name="c", subcore_axis_name="s")

    cp = pltpu.CompilerParams()
    if "needs_layout_passes" in pltpu.CompilerParams.__dataclass_fields__:
        cp = dataclasses.replace(cp, needs_layout_passes=False)

    @functools.partial(
        pl.kernel,
        mesh=mesh,
        compiler_params=cp,
        out_type=(jax.ShapeDtypeStruct((D, B), jnp.float32),
                  jax.ShapeDtypeStruct((D, B), jnp.float32)),
        scratch_types=[
            pltpu.VMEM((bpw,), jnp.int32),
            pltpu.VMEM((bpw,), jnp.int32),
            pltpu.VMEM((16, D, _LANES), jnp.float32),
            pltpu.VMEM((16, D, _LANES), jnp.float32),
            pltpu.VMEM((D, bpw), jnp.float32),
            pltpu.VMEM((D, bpw), jnp.float32),
            pltpu.SemaphoreType.DMA,
            pltpu.SemaphoreType.DMA,
        ],
    )
    def sc_k(ut_hbm, it_hbm, ui_hbm, ii_hbm, uo_hbm, io_hbm,
             uidx_v, iidx_v, ubuf, ibuf, uout, iout,
             usem, isem):
        wid = lax.axis_index("s") * _NC + lax.axis_index("c")
        base = wid * bpw
        iota16 = lax.iota(jnp.int32, 16)
        pltpu.sync_copy(ui_hbm.at[pl.ds(base, bpw)], uidx_v)
        pltpu.sync_copy(ii_hbm.at[pl.ds(base, bpw)], iidx_v)

        @pl.loop(0, bpw // 16)
        def _(c):
            c16 = pl.multiple_of(c * 16, 16)
            uvecs = uidx_v[pl.ds(c16, 16)]
            ivecs = iidx_v[pl.ds(c16, 16)]
            for j in range(16):
                ou = pl.multiple_of((uvecs[j] >> 7) * _LANES, _LANES)
                oi = pl.multiple_of((ivecs[j] >> 7) * _LANES, _LANES)
                pltpu.async_copy(ut_hbm.at[:, pl.ds(ou, _LANES)],
                                 ubuf.at[j], usem)
                pltpu.async_copy(it_hbm.at[:, pl.ds(oi, _LANES)],
                                 ibuf.at[j], isem)
            ulanes = uvecs & (_LANES - 1)
            ilanes = ivecs & (_LANES - 1)
            for j in range(16):
                pltpu.make_async_copy(ut_hbm.at[:, pl.ds(0, _LANES)],
                                      ubuf.at[j], usem).wait()
                pltpu.make_async_copy(it_hbm.at[:, pl.ds(0, _LANES)],
                                      ibuf.at[j], isem).wait()
                j16 = jnp.zeros((16,), jnp.int32) + j
                g16 = jnp.zeros((16,), jnp.int32) + (c16 + j)
                lu = jnp.zeros((16,), jnp.int32) + ulanes[j]
                li = jnp.zeros((16,), jnp.int32) + ilanes[j]
                uvec = plsc.load_gather(ubuf, [j16, iota16, lu])
                ivec = plsc.load_gather(ibuf, [j16, iota16, li])
                plsc.store_scatter(uout, [iota16, g16], uvec)
                plsc.store_scatter(iout, [iota16, g16], ivec)

        pltpu.sync_copy(uout, uo_hbm.at[:, pl.ds(base, bpw)])
        pltpu.sync_copy(iout, io_hbm.at[:, pl.ds(base, bpw)])

    return sc_k(user_t, item_t, user_idx, item_idx)


def _mlp_body(xu_ref, xi_ref, w1u_ref, w1i_ref, b1_ref,
              w2_ref, b2_ref, w3_ref, b3_ref, wf_ref, bf_ref, o_ref):
    hp = jax.lax.Precision.HIGHEST
    h = jnp.dot(w1u_ref[...], xu_ref[...], precision=hp)
    h += jnp.dot(w1i_ref[...], xi_ref[...], precision=hp)
    h = jnp.maximum(h + b1_ref[...], 0.0)
    h = jnp.maximum(jnp.dot(w2_ref[...], h, precision=hp) + b2_ref[...], 0.0)
    h = jnp.maximum(jnp.dot(w3_ref[...], h, precision=hp) + b3_ref[...], 0.0)
    logits = jnp.dot(wf_ref[...], h, precision=hp) + bf_ref[...]
    o_ref[...] = jax.nn.sigmoid(logits)


def _tc_mlp_t(xu, xi, W1, b1, W2, b2, W3, b3, Wf, bf):
    D, B = xu.shape
    blk = 2048
    w1ut = W1[:D].T
    w1it = W1[D:].T
    w2t, w3t, wft = W2.T, W3.T, Wf.T
    b1c, b2c, b3c = b1.reshape(-1, 1), b2.reshape(-1, 1), b3.reshape(-1, 1)
    bfc = bf.reshape(1, 1)

    full = lambda shape: pl.BlockSpec(shape, lambda b: (0, 0))
    out = pl.pallas_call(
        _mlp_body,
        grid=(B // blk,),
        in_specs=[
            pl.BlockSpec((D, blk), lambda b: (0, b)),
            pl.BlockSpec((D, blk), lambda b: (0, b)),
            full(w1ut.shape), full(w1it.shape), full(b1c.shape),
            full(w2t.shape), full(b2c.shape),
            full(w3t.shape), full(b3c.shape),
            full(wft.shape), full(bfc.shape),
        ],
        out_specs=pl.BlockSpec((1, blk), lambda b: (0, b)),
        out_shape=jax.ShapeDtypeStruct((1, B), jnp.float32),
        compiler_params=pltpu.CompilerParams(
            dimension_semantics=("parallel",)),
    )(xu, xi, w1ut, w1it, b1c, w2t, b2c, w3t, b3c, wft, bfc)
    return out.reshape(B, 1)


def kernel(user_input, item_input, user_table, item_table,
           W1, b1, W2, b2, W3, b3, Wf, bf):
    xu, xi = _sc_gather_t(user_table.T, item_table.T, user_input, item_input)
    return _tc_mlp_t(xu, xi, W1, b1, W2, b2, W3, b3, Wf, bf)


# speedup vs baseline: 15.2236x; 1.2461x over previous
import dataclasses
import functools

import jax
import jax.numpy as jnp
from jax import lax
from jax.experimental import pallas as pl
from jax.experimental.pallas import tpu as pltpu
from jax.experimental.pallas import tpu_sc as plsc

_NC = 2
_NS = 16
_NW = _NC * _NS
_NBUF = 8
_LANES = 128


def _sc_gather_t(user_t, item_t, user_idx, item_idx):
    D, V = user_t.shape
    B = user_idx.shape[0]
    bpw = B // _NW

    mesh = plsc.VectorSubcoreMesh(core_axis_name="c", subcore_axis_name="s")

    cp = pltpu.CompilerParams()
    if "needs_layout_passes" in pltpu.CompilerParams.__dataclass_fields__:
        cp = dataclasses.replace(cp, needs_layout_passes=False)

    @functools.partial(
        pl.kernel,
        mesh=mesh,
        compiler_params=cp,
        out_type=(jax.ShapeDtypeStruct((D, B), jnp.float32),
                  jax.ShapeDtypeStruct((D, B), jnp.float32)),
        scratch_types=[
            pltpu.VMEM((bpw,), jnp.int32),
            pltpu.VMEM((bpw,), jnp.int32),
            pltpu.VMEM((32, D, _LANES), jnp.float32),
            pltpu.VMEM((D, bpw), jnp.float32),
            pltpu.VMEM((D, bpw), jnp.float32),
            pltpu.SemaphoreType.DMA,
        ],
    )
    def sc_k(ut_hbm, it_hbm, ui_hbm, ii_hbm, uo_hbm, io_hbm,
             uidx_v, iidx_v, buf, uout, iout, sem):
        wid = lax.axis_index("s") * _NC + lax.axis_index("c")
        base = wid * bpw
        iota16 = lax.iota(jnp.int32, 16)
        pltpu.sync_copy(ui_hbm.at[pl.ds(base, bpw)], uidx_v)
        pltpu.sync_copy(ii_hbm.at[pl.ds(base, bpw)], iidx_v)
        nchunks = bpw // 16

        def run_pass(t_hbm, idx_v, out):
            @pl.loop(0, nchunks + 1)
            def _(c):
                @pl.when(c < nchunks)
                def _():
                    c16 = pl.multiple_of(c * 16, 16)
                    vecs = idx_v[pl.ds(c16, 16)]
                    s0 = lax.rem(c, 2) * 16
                    for j in range(16):
                        o = pl.multiple_of((vecs[j] >> 7) * _LANES, _LANES)
                        pltpu.async_copy(t_hbm.at[:, pl.ds(o, _LANES)],
                                         buf.at[s0 + j], sem)

                @pl.when(c > 0)
                def _():
                    cm16 = pl.multiple_of((c - 1) * 16, 16)
                    vecs = idx_v[pl.ds(cm16, 16)]
                    lanes = vecs & (_LANES - 1)
                    s0 = lax.rem(c - 1, 2) * 16
                    for j in range(16):
                        pltpu.make_async_copy(
                            t_hbm.at[:, pl.ds(0, _LANES)],
                            buf.at[s0 + j], sem).wait()
                        s16 = jnp.zeros((16,), jnp.int32) + (s0 + j)
                        g16 = jnp.zeros((16,), jnp.int32) + (cm16 + j)
                        lv = jnp.zeros((16,), jnp.int32) + lanes[j]
                        vec = plsc.load_gather(buf, [s16, iota16, lv])
                        plsc.store_scatter(out, [iota16, g16], vec)

        run_pass(ut_hbm, uidx_v, uout)
        run_pass(it_hbm, iidx_v, iout)
        pltpu.sync_copy(uout, uo_hbm.at[:, pl.ds(base, bpw)])
        pltpu.sync_copy(iout, io_hbm.at[:, pl.ds(base, bpw)])

    return sc_k(user_t, item_t, user_idx, item_idx)


def _mlp_body(xu_ref, xi_ref, w1u_ref, w1i_ref, b1_ref,
              w2_ref, b2_ref, w3_ref, b3_ref, wf_ref, bf_ref, o_ref):
    hp = jax.lax.Precision.HIGHEST
    h = jnp.dot(w1u_ref[...], xu_ref[...], precision=hp)
    h += jnp.dot(w1i_ref[...], xi_ref[...], precision=hp)
    h = jnp.maximum(h + b1_ref[...], 0.0)
    h = jnp.maximum(jnp.dot(w2_ref[...], h, precision=hp) + b2_ref[...], 0.0)
    h = jnp.maximum(jnp.dot(w3_ref[...], h, precision=hp) + b3_ref[...], 0.0)
    logits = jnp.dot(wf_ref[...], h, precision=hp) + bf_ref[...]
    o_ref[...] = jax.nn.sigmoid(logits)


def _tc_mlp_t(xu, xi, W1, b1, W2, b2, W3, b3, Wf, bf):
    D, B = xu.shape
    blk = 2048
    w1ut = W1[:D].T
    w1it = W1[D:].T
    w2t, w3t, wft = W2.T, W3.T, Wf.T
    b1c, b2c, b3c = b1.reshape(-1, 1), b2.reshape(-1, 1), b3.reshape(-1, 1)
    bfc = bf.reshape(1, 1)

    full = lambda shape: pl.BlockSpec(shape, lambda b: (0, 0))
    out = pl.pallas_call(
        _mlp_body,
        grid=(B // blk,),
        in_specs=[
            pl.BlockSpec((D, blk), lambda b: (0, b)),
            pl.BlockSpec((D, blk), lambda b: (0, b)),
            full(w1ut.shape), full(w1it.shape), full(b1c.shape),
            full(w2t.shape), full(b2c.shape),
            full(w3t.shape), full(b3c.shape),
            full(wft.shape), full(bfc.shape),
        ],
        out_specs=pl.BlockSpec((1, blk), lambda b: (0, b)),
        out_shape=jax.ShapeDtypeStruct((1, B), jnp.float32),
        compiler_params=pltpu.CompilerParams(
            dimension_semantics=("parallel",)),
    )(xu, xi, w1ut, w1it, b1c, w2t, b2c, w3t, b3c, wft, bfc)
    return out.reshape(B, 1)


def kernel(user_input, item_input, user_table, item_table,
           W1, b1, W2, b2, W3, b3, Wf, bf):
    xu, xi = _sc_gather_t(user_table.T, item_table.T, user_input, item_input)
    return _tc_mlp_t(xu, xi, W1, b1, W2, b2, W3, b3, Wf, bf)
